# baseline (speedup 1.0000x reference)
import functools

import jax
import jax.numpy as jnp
from jax import lax
from jax.experimental import pallas as pl
from jax.experimental.pallas import tpu as pltpu
from jax.experimental.pallas import tpu_sc as plsc

N, P, C = 4, 8192, 1024
R = N * P
NW = 32
RPW = R // NW
CH = 16
NCH = RPW // CH
VPR = C // 16


def _body(x_hbm, idx_hbm, mask_hbm, out_hbm, idx_v, gat_v, x_v, m_v, sem):
    wid = lax.axis_index("s") * 2 + lax.axis_index("c")
    pltpu.sync_copy(idx_hbm.at[wid], idx_v)

    def chunk(ch, _):
        base = wid * RPW + ch * CH
        gcp = pltpu.async_copy(x_hbm.at[idx_v.at[ch]], gat_v, sem)
        pltpu.sync_copy(x_hbm.at[pl.ds(base, CH)], x_v)
        pltpu.sync_copy(mask_hbm.at[pl.ds(base, CH)], m_v)
        gcp.wait()

        def row(r, _):
            def col(v, _):
                c = v * 16
                m = m_v[r, pl.ds(c, 16)]
                g = gat_v[r, pl.ds(c, 16)]
                xv = x_v[r, pl.ds(c, 16)]
                x_v[r, pl.ds(c, 16)] = jnp.where(m != 0, g, xv)
                return 0

            return lax.fori_loop(0, VPR, col, 0)

        lax.fori_loop(0, CH, row, 0)
        pltpu.sync_copy(x_v, out_hbm.at[pl.ds(base, CH)])
        return 0

    lax.fori_loop(0, NCH, chunk, 0)


@jax.jit
def _run(x_flat, idx_w, mask_i32):
    mesh = plsc.VectorSubcoreMesh(core_axis_name="c", subcore_axis_name="s")
    k = functools.partial(
        pl.kernel,
        mesh=mesh,
        out_type=jax.ShapeDtypeStruct((R, C), jnp.float32),
        scratch_types=[
            pltpu.VMEM((NCH, CH), jnp.int32),
            pltpu.VMEM((CH, C), jnp.float32),
            pltpu.VMEM((CH, C), jnp.float32),
            pltpu.VMEM((CH, C), jnp.int32),
            pltpu.SemaphoreType.DMA,
        ],
    )(_body)
    return k(x_flat, idx_w, mask_i32)


def kernel(x, idx, drop_mask):
    x_flat = x.reshape(R, C)
    idx_w = idx.astype(jnp.int32).reshape(NW, NCH, CH)
    mask_i32 = drop_mask.reshape(R, C).astype(jnp.int32)
    out = _run(x_flat, idx_w, mask_i32)
    return out.reshape(x.shape)
